# Initial kernel scaffold; baseline (speedup 1.0000x reference)
import jax
import jax.numpy as jnp
from jax.experimental import pallas as pl


def kernel(shape, color, pose, shape_table, color_table, W, b):
    raise NotImplementedError("write your pallas kernel here")


# speedup vs baseline: 2.4917x; 2.4917x over previous
import functools

import jax
import jax.numpy as jnp
from jax import lax
from jax.experimental import pallas as pl
from jax.experimental.pallas import tpu as pltpu
from jax.experimental.pallas import tpu_sc as plsc

B = 4096
N = 200
C = 64
R = N * B

NC = 2
NS = 16
NW = NC * NS
ROWS_PER_W = R // NW
SUB = 128
CHUNK = 2 * SUB
CHUNKS_PER_W = ROWS_PER_W // CHUNK


def _sc_gather_sum(idx_s, idx_c, shape_table, color_table):
    mesh = plsc.VectorSubcoreMesh(core_axis_name="c", subcore_axis_name="s")

    @functools.partial(
        pl.kernel,
        out_type=jax.ShapeDtypeStruct((R, C), jnp.float32),
        mesh=mesh,
        scratch_types=[
            pltpu.VMEM((2, SUB), jnp.int32),
            pltpu.VMEM((2, SUB), jnp.int32),
            pltpu.VMEM((CHUNK, C), jnp.float32),
            pltpu.VMEM((CHUNK, C), jnp.float32),
            pltpu.VMEM((CHUNK, C), jnp.float32),
            pltpu.SemaphoreType.DMA,
        ],
        compiler_params=pltpu.CompilerParams(use_tc_tiling_on_sc=False),
    )
    def k(idx_s_hbm, idx_c_hbm, stab_hbm, ctab_hbm, out_hbm,
          idxs_v, idxc_v, rows_s, rows_c, out_v, sem):
        wid = lax.axis_index("s") * NC + lax.axis_index("c")
        base_irow = wid * (ROWS_PER_W // SUB)
        base_row = wid * ROWS_PER_W

        @pl.loop(0, CHUNKS_PER_W)
        def _(t):
            irow = base_irow + t * 2
            goff = base_row + t * CHUNK
            pltpu.sync_copy(idx_s_hbm.at[pl.ds(irow, 2)], idxs_v)
            pltpu.sync_copy(idx_c_hbm.at[pl.ds(irow, 2)], idxc_v)
            cps = [
                pltpu.async_copy(stab_hbm.at[idxs_v.at[0]],
                                 rows_s.at[pl.ds(0, SUB)], sem),
                pltpu.async_copy(stab_hbm.at[idxs_v.at[1]],
                                 rows_s.at[pl.ds(SUB, SUB)], sem),
                pltpu.async_copy(ctab_hbm.at[idxc_v.at[0]],
                                 rows_c.at[pl.ds(0, SUB)], sem),
                pltpu.async_copy(ctab_hbm.at[idxc_v.at[1]],
                                 rows_c.at[pl.ds(SUB, SUB)], sem),
            ]
            for cp in cps:
                cp.wait()

            @pl.loop(0, CHUNK)
            def _(i):
                for j in range(C // 16):
                    sl = pl.ds(j * 16, 16)
                    out_v[i, sl] = rows_s[i, sl] + rows_c[i, sl]

            pltpu.sync_copy(out_v, out_hbm.at[pl.ds(goff, CHUNK)])

    return k(idx_s, idx_c, shape_table, color_table)


def _tc_pose_add(pose_t, sum3d, W, b2):
    BB = 512

    def body(pose_ref, sum_ref, w_ref, b_ref, out_ref):
        mm = jnp.dot(pose_ref[0], w_ref[...],
                     preferred_element_type=jnp.float32)
        out_ref[...] = (mm + b_ref[...] + sum_ref[0])[None]

    return pl.pallas_call(
        body,
        grid=(N, B // BB),
        in_specs=[
            pl.BlockSpec((1, BB, 16), lambda i, j: (i, j, 0)),
            pl.BlockSpec((1, BB, C), lambda i, j: (i, j, 0)),
            pl.BlockSpec((16, C), lambda i, j: (0, 0)),
            pl.BlockSpec((1, C), lambda i, j: (0, 0)),
        ],
        out_specs=pl.BlockSpec((1, BB, C), lambda i, j: (i, j, 0)),
        out_shape=jax.ShapeDtypeStruct((N, B, C), jnp.float32),
    )(pose_t, sum3d, W, b2)


def kernel(shape, color, pose, shape_table, color_table, W, b):
    idx_s = shape.astype(jnp.int32).T.reshape(R // SUB, SUB)
    idx_c = color.astype(jnp.int32).T.reshape(R // SUB, SUB)
    sum_emb = _sc_gather_sum(idx_s, idx_c, shape_table, color_table)
    pose_t = pose.transpose(1, 0, 2)
    out = _tc_pose_add(pose_t, sum_emb.reshape(N, B, C), W, b.reshape(1, C))
    return out


# speedup vs baseline: 5.2639x; 2.1125x over previous
import functools

import jax
import jax.numpy as jnp
from jax import lax
from jax.experimental import pallas as pl
from jax.experimental.pallas import tpu as pltpu
from jax.experimental.pallas import tpu_sc as plsc

B = 4096
N = 200
C = 64
R = N * B

NC = 2
NS = 16
NW = NC * NS
ROWS_PER_W = R // NW
CHUNK = 128
CHUNKS_PER_W = ROWS_PER_W // CHUNK


def _sc_gather_sum(idx_s, idx_c, shape_table, color_table):
    mesh = plsc.VectorSubcoreMesh(core_axis_name="c", subcore_axis_name="s")

    @functools.partial(
        pl.kernel,
        out_type=jax.ShapeDtypeStruct((R // 2, 2 * C), jnp.float32),
        mesh=mesh,
        scratch_types=[
            pltpu.VMEM((ROWS_PER_W,), jnp.int32),
            pltpu.VMEM((ROWS_PER_W,), jnp.int32),
            pltpu.VMEM((2, CHUNK, C), jnp.float32),
            pltpu.VMEM((2, CHUNK, C), jnp.float32),
            pltpu.VMEM((2, CHUNK // 2, 2 * C), jnp.float32),
            pltpu.SemaphoreType.DMA,
            pltpu.SemaphoreType.DMA,
            pltpu.SemaphoreType.DMA,
            pltpu.SemaphoreType.DMA,
        ],
        compiler_params=pltpu.CompilerParams(use_tc_tiling_on_sc=False),
    )
    def k(idx_s_hbm, idx_c_hbm, stab_hbm, ctab_hbm, out_hbm,
          idxs_v, idxc_v, rows_s, rows_c, out_v, gs0, gs1, ws0, ws1):
        gsem = (gs0, gs1)
        wsem = (ws0, ws1)
        wid = lax.axis_index("s") * NC + lax.axis_index("c")
        base = wid * ROWS_PER_W
        obase = wid * (ROWS_PER_W // 2)

        pltpu.sync_copy(idx_s_hbm.at[pl.ds(base, ROWS_PER_W)], idxs_v)
        pltpu.sync_copy(idx_c_hbm.at[pl.ds(base, ROWS_PER_W)], idxc_v)

        def fire(t, p):
            isl = idxs_v.at[pl.ds(t * CHUNK, CHUNK)]
            icl = idxc_v.at[pl.ds(t * CHUNK, CHUNK)]
            pltpu.async_copy(stab_hbm.at[isl], rows_s.at[p], gsem[p])
            pltpu.async_copy(ctab_hbm.at[icl], rows_c.at[p], gsem[p])

        def drain_gather(p):
            pltpu.make_async_copy(stab_hbm.at[pl.ds(0, CHUNK)],
                                  rows_s.at[p], gsem[p]).wait()
            pltpu.make_async_copy(ctab_hbm.at[pl.ds(0, CHUNK)],
                                  rows_c.at[p], gsem[p]).wait()

        def drain_write(p):
            pltpu.make_async_copy(out_hbm.at[pl.ds(0, CHUNK // 2)],
                                  out_v.at[p], wsem[p]).wait()

        fire(0, 0)

        @pl.loop(0, CHUNKS_PER_W // 2)
        def _(g):
            for p in (0, 1):
                t = g * 2 + p

                @pl.when(t < CHUNKS_PER_W - 1)
                def _():
                    fire(t + 1, 1 - p)

                drain_gather(p)

                @pl.when(t >= 2)
                def _():
                    drain_write(p)

                @pl.loop(0, CHUNK // 2)
                def _(i):
                    for q in (0, 1):
                        for j in range(C // 16):
                            sl = pl.ds(q * C + j * 16, 16)
                            ssl = pl.ds(j * 16, 16)
                            out_v[p, i, sl] = (rows_s[p, i * 2 + q, ssl]
                                               + rows_c[p, i * 2 + q, ssl])

                pltpu.async_copy(
                    out_v.at[p],
                    out_hbm.at[pl.ds(obase + t * (CHUNK // 2), CHUNK // 2)],
                    wsem[p])

        drain_write(0)
        drain_write(1)

    return k(idx_s, idx_c, shape_table, color_table)


def _tc_pose_add(pose_p, sum_p, W2, b2):
    NB = 8
    KB = 512

    def body(pose_ref, sum_ref, w_ref, b_ref, out_ref):
        x = pose_ref[...].reshape(NB * KB, 32)
        mm = jnp.dot(x, w_ref[...], preferred_element_type=jnp.float32)
        y = mm + b_ref[...] + sum_ref[...].reshape(NB * KB, 2 * C)
        out_ref[...] = y.reshape(NB, KB, 2 * C)

    return pl.pallas_call(
        body,
        grid=(N // NB, (B // 2) // KB),
        in_specs=[
            pl.BlockSpec((NB, KB, 32), lambda i, j: (i, j, 0)),
            pl.BlockSpec((NB, KB, 2 * C), lambda i, j: (i, j, 0)),
            pl.BlockSpec((32, 2 * C), lambda i, j: (0, 0)),
            pl.BlockSpec((1, 2 * C), lambda i, j: (0, 0)),
        ],
        out_specs=pl.BlockSpec((NB, KB, 2 * C), lambda i, j: (i, j, 0)),
        out_shape=jax.ShapeDtypeStruct((N, B // 2, 2 * C), jnp.float32),
    )(pose_p, sum_p, W2, b2)


def kernel(shape, color, pose, shape_table, color_table, W, b):
    idx_s = shape.astype(jnp.int32).T.reshape(R)
    idx_c = color.astype(jnp.int32).T.reshape(R)
    sum_emb = _sc_gather_sum(idx_s, idx_c, shape_table, color_table)

    zero = jnp.zeros((16, C), dtype=W.dtype)
    W2 = jnp.concatenate(
        [jnp.concatenate([W, zero], axis=1),
         jnp.concatenate([zero, W], axis=1)], axis=0)
    b2 = jnp.concatenate([b, b]).reshape(1, 2 * C)
    pose_p = pose.transpose(1, 0, 2).reshape(N, B // 2, 32)
    out = _tc_pose_add(pose_p, sum_emb.reshape(N, B // 2, 2 * C), W2, b2)
    return out.reshape(N, B, C)
